# baseline (speedup 1.0000x reference)
import functools

import jax
import jax.numpy as jnp
import numpy as np
from jax import lax
from jax.experimental import pallas as pl
from jax.experimental.pallas import tpu as pltpu
from jax.experimental.pallas import tpu_sc as plsc

_R = 0.2


def _sqdist(a, b):
    d = -2.0 * jnp.einsum('bsd,bnd->bsn', a, b)
    d = d + jnp.sum(a * a, axis=-1)[:, :, None]
    d = d + jnp.sum(b * b, axis=-1)[:, None, :]
    return d


def _fps_body(xyz_ref, inds_ref, newxyz_ref, neg2_ref, sb_ref, sa_ref,
              dists_ref, *, B, R, LC, N, npoint):
    jiota = (lax.broadcasted_iota(jnp.int32, (R, LC), 0) * LC
             + lax.broadcasted_iota(jnp.int32, (R, LC), 1))
    kiota = lax.broadcasted_iota(jnp.int32, (8, npoint), 1)
    for b in range(B):
        dists_ref[b] = jnp.where(jiota < N, jnp.float32(1e10), jnp.float32(-1.0))
        x = xyz_ref[0, b]
        y = xyz_ref[1, b]
        z = xyz_ref[2, b]
        sb = (x * x + y * y) + z * z
        sb_ref[b] = jnp.where(jiota < N, sb, jnp.float32(1e10))
        for d in range(3):
            neg2_ref[d, b] = jnp.float32(-2.0) * xyz_ref[d, b]

    def body(i, fars):
        newfars = []
        for b in range(B):
            far_b = fars[b]
            inds_ref[b] = jnp.where(kiota == i, far_b, inds_ref[b])
            sel = jiota == far_b
            x = xyz_ref[0, b]
            y = xyz_ref[1, b]
            z = xyz_ref[2, b]
            cx = jnp.sum(jnp.where(sel, x, 0.0))
            cy = jnp.sum(jnp.where(sel, y, 0.0))
            cz = jnp.sum(jnp.where(sel, z, 0.0))
            newxyz_ref[0, b] = jnp.where(kiota == i, cx, newxyz_ref[0, b])
            newxyz_ref[1, b] = jnp.where(kiota == i, cy, newxyz_ref[1, b])
            newxyz_ref[2, b] = jnp.where(kiota == i, cz, newxyz_ref[2, b])
            sa_b = (cx * cx + cy * cy) + cz * cz
            sa_ref[b] = jnp.where(kiota == i, sa_b, sa_ref[b])
            dx = x - cx
            dy = y - cy
            dz = z - cz
            d = (dx * dx + dy * dy) + dz * dz
            dmin = jnp.minimum(dists_ref[b], d)
            dists_ref[b] = dmin
            m = jnp.max(dmin)
            far_n = jnp.min(jnp.where(dmin == m, jiota, jnp.int32(2 ** 30)))
            newfars.append(far_n)
        return tuple(newfars)

    lax.fori_loop(0, npoint, body, tuple(jnp.int32(0) for _ in range(B)),
                  unroll=False)


def _fps(xyz, npoint, interpret=False):
    B, N, _ = xyz.shape
    R = 8
    LC = -(-N // (R * 128)) * 128
    Npad = R * LC
    xyzp = jnp.moveaxis(xyz, -1, 0)
    xyzp = jnp.pad(xyzp, ((0, 0), (0, 0), (0, Npad - N)))
    xyzp = xyzp.reshape(3, B, R, LC)
    inds, newxyz, neg2, sb, sa = pl.pallas_call(
        functools.partial(_fps_body, B=B, R=R, LC=LC, N=N, npoint=npoint),
        out_shape=[
            jax.ShapeDtypeStruct((B, 8, npoint), jnp.int32),
            jax.ShapeDtypeStruct((3, B, 8, npoint), jnp.float32),
            jax.ShapeDtypeStruct((3, B, R, LC), jnp.float32),
            jax.ShapeDtypeStruct((B, R, LC), jnp.float32),
            jax.ShapeDtypeStruct((B, 8, npoint), jnp.float32),
        ],
        scratch_shapes=[pltpu.VMEM((B, R, LC), jnp.float32)],
        interpret=interpret,
    )(xyzp)
    new_xyz = jnp.moveaxis(newxyz[:, :, 0, :], 0, -1)
    return (inds[:, 0, :], new_xyz, neg2.reshape(3, B, Npad),
            sb.reshape(B, Npad), sa[:, 0, :])

def _ball_group_sc(neg2, sb, cxyz, sa, table, radius, S, nsample, D):
    B = sb.shape[0]
    Npad = sb.shape[1]
    NW = 32
    SPT = (B * S) // NW
    SPTB = max(SPT, 16)
    TPB = S // SPT
    r2 = jnp.float32(radius * radius)
    NS16 = nsample // 16
    NT = Npad // 16

    def _per_tile(v):
        v = v.reshape(NW, SPT)
        return jnp.pad(v, ((0, 0), (0, SPTB - SPT)))

    mesh = plsc.VectorSubcoreMesh(core_axis_name="c", subcore_axis_name="s")

    @functools.partial(
        pl.kernel, mesh=mesh,
        out_type=jax.ShapeDtypeStruct((B * S, nsample, D), jnp.float32),
        scratch_types=[
            pltpu.VMEM((Npad,), jnp.float32),
            pltpu.VMEM((Npad,), jnp.float32),
            pltpu.VMEM((Npad,), jnp.float32),
            pltpu.VMEM((Npad,), jnp.float32),
            pltpu.VMEM((SPTB,), jnp.float32),
            pltpu.VMEM((SPTB,), jnp.float32),
            pltpu.VMEM((SPTB,), jnp.float32),
            pltpu.VMEM((SPTB,), jnp.float32),
            pltpu.VMEM((nsample + 16,), jnp.int32),
            pltpu.VMEM((2, nsample), jnp.int32),
            pltpu.VMEM((2, nsample, D), jnp.float32),
            pltpu.SemaphoreType.DMA,
            pltpu.SemaphoreType.DMA,
        ],
        compiler_params=pltpu.CompilerParams(needs_layout_passes=False,
                                             use_tc_tiling_on_sc=False),
    )
    def bq(nx2_h, ny2_h, nz2_h, sb_h, ax_h, ay_h, az_h, sa_h, table_h, out_h,
           nx2_v, ny2_v, nz2_v, sb_v, ax_v, ay_v, az_v, sa_v,
           idx_v, gidx_v, rows_v, gsem0, osem0):
        l16 = lax.broadcasted_iota(jnp.int32, (16,), 0)
        wid = lax.axis_index("s") * 2 + lax.axis_index("c")
        b = wid // TPB
        pltpu.sync_copy(nx2_h.at[b], nx2_v)
        pltpu.sync_copy(ny2_h.at[b], ny2_v)
        pltpu.sync_copy(nz2_h.at[b], nz2_v)
        pltpu.sync_copy(sb_h.at[b], sb_v)
        pltpu.sync_copy(ax_h.at[wid], ax_v)
        pltpu.sync_copy(ay_h.at[wid], ay_v)
        pltpu.sync_copy(az_h.at[wid], az_v)
        pltpu.sync_copy(sa_h.at[wid], sa_v)

        def scan_center(c):
            cl = lax.rem(c, 16)
            gbase = c - cl
            lsel = l16 == cl
            ax_c = jnp.sum(jnp.where(lsel, ax_v[pl.ds(gbase, 16)], 0.0), axis=0)
            ay_c = jnp.sum(jnp.where(lsel, ay_v[pl.ds(gbase, 16)], 0.0), axis=0)
            az_c = jnp.sum(jnp.where(lsel, az_v[pl.ds(gbase, 16)], 0.0), axis=0)
            sa_c = jnp.sum(jnp.where(lsel, sa_v[pl.ds(gbase, 16)], 0.0), axis=0)

            def scan_body(t, cnt):
                off = t * 16
                dot2 = (ax_c * nx2_v[pl.ds(off, 16)]
                        + ay_c * ny2_v[pl.ds(off, 16)])
                dot2 = dot2 + az_c * nz2_v[pl.ds(off, 16)]
                d = dot2 + sa_c
                d = d + sb_v[pl.ds(off, 16)]
                inr = jnp.logical_not(d > r2)

                @pl.when(cnt < nsample)
                def _():
                    plsc.store_compressed(idx_v.at[pl.ds(cnt, 16)],
                                          off + l16, mask=inr)

                pc = plsc.all_reduce_population_count(inr)
                return cnt + jnp.max(pc, axis=0)

            cnt = lax.fori_loop(0, NT, scan_body, jnp.int32(0), unroll=4)
            t_eff = jnp.minimum(cnt, nsample)
            first = jnp.sum(jnp.where(l16 == 0, idx_v[pl.ds(0, 16)], 0), axis=0)
            for g in range(NS16):
                slot = g * 16 + l16
                vg = idx_v[pl.ds(g * 16, 16)]
                gidx_v[0, pl.ds(g * 16, 16)] = (
                    jnp.where(slot < t_eff, vg, first) + b * Npad)

        def center_body(c, carry):
            scan_center(c)
            pltpu.async_copy(table_h.at[gidx_v.at[0]], rows_v.at[0],
                             gsem0).wait()
            pltpu.sync_copy(rows_v.at[0], out_h.at[wid * SPT + c])
            return carry

        lax.fori_loop(0, SPT, center_body, jnp.int32(0), unroll=False)

    return bq(neg2[0], neg2[1], neg2[2], sb,
              _per_tile(cxyz[0]), _per_tile(cxyz[1]), _per_tile(cxyz[2]),
              _per_tile(sa), table)


def _gather(points, idx):
    B = idx.shape[0]
    C = points.shape[-1]
    flat = jnp.take_along_axis(points, idx.reshape(B, -1, 1), axis=1)
    return flat.reshape(idx.shape + (C,))

def _ball(radius, nsample, xyz, new_xyz):
    B, N, _ = xyz.shape
    sqr = _sqdist(new_xyz, xyz)
    mask = sqr <= radius * radius
    c = jnp.cumsum(mask.astype(jnp.int32), axis=-1)
    k = jnp.arange(nsample, dtype=jnp.int32)
    idx = jnp.sum((c[:, :, None, :] <= k[None, None, :, None]).astype(jnp.int32), axis=-1)
    first = idx[:, :, 0:1]
    idx = jnp.where(idx == N, jnp.broadcast_to(first, idx.shape), idx)
    return idx

def _cbr(x, layer):
    W, b, gm, bt = layer
    y = jnp.einsum('...i,io->...o', x, W) + b
    axes = tuple(range(y.ndim - 1))
    mean = jnp.mean(y, axis=axes)
    var = jnp.var(y, axis=axes)
    y = (y - mean) / jnp.sqrt(var + 1e-5) * gm + bt
    return jax.nn.relu(y)

def _sa(xyz, features, npoint, radius, nsample, layers):
    B, N, _ = xyz.shape
    inds, new_xyz, neg2, sb, sa = _fps(xyz, npoint)
    Npad = sb.shape[1]
    C = 3 + (features.shape[-1] if features is not None else 0)
    D = -(-C // 16) * 16
    table = xyz if features is None else jnp.concatenate([xyz, features], -1)
    table = jnp.pad(table, ((0, 0), (0, Npad - N), (0, D - C)))
    table = table.reshape(B * Npad, D)
    cxyz = jnp.moveaxis(new_xyz, -1, 0)
    rows = _ball_group_sc(neg2, sb, cxyz, sa, table, radius,
                          npoint, nsample, D)
    rows = rows.reshape(B, npoint, nsample, D)
    gx = (rows[..., :3] - new_xyz[:, :, None, :]) / radius
    if features is not None:
        g = jnp.concatenate([gx, rows[..., 3:C]], axis=-1)
    else:
        g = gx
    for layer in layers:
        g = _cbr(g, layer)
    return new_xyz, jnp.max(g, axis=2), inds

def _fp(xyz1, xyz2, feat1, feat2, layers):
    d = _sqdist(xyz1, xyz2)
    negd, idx = lax.top_k(-d, 3)
    dist_recip = 1.0 / (-negd + 1e-8)
    weight = dist_recip / jnp.sum(dist_recip, axis=-1, keepdims=True)
    interp = jnp.sum(_gather(feat2, idx) * weight[..., None], axis=2)
    g = jnp.concatenate([interp, feat1], axis=-1)
    for layer in layers:
        g = _cbr(g, layer)
    return g


def kernel(pointcloud, params):
    xyz = pointcloud[..., 0:3]
    features = pointcloud[..., 3:]
    sa1_xyz, sa1_f, sa1_inds = _sa(xyz, features, 512, _R, 64, params['sa1'])
    sa2_xyz, sa2_f, sa2_inds = _sa(sa1_xyz, sa1_f, 256, 2 * _R, 32, params['sa2'])
    sa3_xyz, sa3_f, sa3_inds = _sa(sa2_xyz, sa2_f, 64, 4 * _R, 16, params['sa3'])
    sa4_xyz, sa4_f, sa4_inds = _sa(sa3_xyz, sa3_f, 16, 8 * _R, 16, params['sa4'])
    fp1_f = _fp(sa3_xyz, sa4_xyz, sa3_f, sa4_f, params['fp1'])
    fp2_f = _fp(sa2_xyz, sa3_xyz, sa2_f, fp1_f, params['fp2'])
    fp2_inds = sa1_inds[:, :fp2_f.shape[1]]
    return fp2_f, sa2_xyz, fp2_inds


# speedup vs baseline: 1.0205x; 1.0205x over previous
import functools

import jax
import jax.numpy as jnp
from jax import lax
from jax.experimental import pallas as pl
from jax.experimental.pallas import tpu as pltpu
from jax.experimental.pallas import tpu_sc as plsc

_R = 0.2


def _sqdist(a, b):
    d = -2.0 * jnp.einsum('bsd,bnd->bsn', a, b)
    d = d + jnp.sum(a * a, axis=-1)[:, :, None]
    d = d + jnp.sum(b * b, axis=-1)[:, None, :]
    return d


def _fps_body(xyz_ref, inds_ref, newxyz_ref, neg2_ref, sb_ref, sa_ref,
              dists_ref, *, B, R, LC, N, npoint):
    jiota = (lax.broadcasted_iota(jnp.int32, (R, LC), 0) * LC
             + lax.broadcasted_iota(jnp.int32, (R, LC), 1))
    kiota = lax.broadcasted_iota(jnp.int32, (8, npoint), 1)
    for b in range(B):
        dists_ref[b] = jnp.where(jiota < N, jnp.float32(1e10), jnp.float32(-1.0))
        x = xyz_ref[0, b]
        y = xyz_ref[1, b]
        z = xyz_ref[2, b]
        sb = (x * x + y * y) + z * z
        sb_ref[b] = jnp.where(jiota < N, sb, jnp.float32(1e10))
        for d in range(3):
            neg2_ref[d, b] = jnp.float32(-2.0) * xyz_ref[d, b]

    def body(i, fars):
        newfars = []
        for b in range(B):
            far_b = fars[b]
            inds_ref[b] = jnp.where(kiota == i, far_b, inds_ref[b])
            sel = jiota == far_b
            x = xyz_ref[0, b]
            y = xyz_ref[1, b]
            z = xyz_ref[2, b]
            cx = jnp.sum(jnp.where(sel, x, 0.0))
            cy = jnp.sum(jnp.where(sel, y, 0.0))
            cz = jnp.sum(jnp.where(sel, z, 0.0))
            newxyz_ref[0, b] = jnp.where(kiota == i, cx, newxyz_ref[0, b])
            newxyz_ref[1, b] = jnp.where(kiota == i, cy, newxyz_ref[1, b])
            newxyz_ref[2, b] = jnp.where(kiota == i, cz, newxyz_ref[2, b])
            sa_b = (cx * cx + cy * cy) + cz * cz
            sa_ref[b] = jnp.where(kiota == i, sa_b, sa_ref[b])
            dx = x - cx
            dy = y - cy
            dz = z - cz
            d = (dx * dx + dy * dy) + dz * dz
            dmin = jnp.minimum(dists_ref[b], d)
            dists_ref[b] = dmin
            m = jnp.max(dmin)
            far_n = jnp.min(jnp.where(dmin == m, jiota, jnp.int32(2 ** 30)))
            newfars.append(far_n)
        return tuple(newfars)

    lax.fori_loop(0, npoint, body, tuple(jnp.int32(0) for _ in range(B)),
                  unroll=False)


def _fps(xyz, npoint, interpret=False):
    B, N, _ = xyz.shape
    R = 8
    LC = -(-N // (R * 128)) * 128
    Npad = R * LC
    xyzp = jnp.moveaxis(xyz, -1, 0)
    xyzp = jnp.pad(xyzp, ((0, 0), (0, 0), (0, Npad - N)))
    xyzp = xyzp.reshape(3, B, R, LC)
    inds, newxyz, neg2, sb, sa = pl.pallas_call(
        functools.partial(_fps_body, B=B, R=R, LC=LC, N=N, npoint=npoint),
        out_shape=[
            jax.ShapeDtypeStruct((B, 8, npoint), jnp.int32),
            jax.ShapeDtypeStruct((3, B, 8, npoint), jnp.float32),
            jax.ShapeDtypeStruct((3, B, R, LC), jnp.float32),
            jax.ShapeDtypeStruct((B, R, LC), jnp.float32),
            jax.ShapeDtypeStruct((B, 8, npoint), jnp.float32),
        ],
        scratch_shapes=[pltpu.VMEM((B, R, LC), jnp.float32)],
        interpret=interpret,
    )(xyzp)
    new_xyz = jnp.moveaxis(newxyz[:, :, 0, :], 0, -1)
    return (inds[:, 0, :], new_xyz, neg2.reshape(3, B, Npad),
            sb.reshape(B, Npad), sa[:, 0, :])

def _ball_group_sc(neg2, sb, cxyz, sa, table, radius, S, nsample, D):
    B = sb.shape[0]
    Npad = sb.shape[1]
    NW = 32
    SPT = (B * S) // NW
    SPTB = max(SPT, 16)
    TPB = S // SPT
    r2 = jnp.float32(radius * radius)
    NS16 = nsample // 16
    NT = Npad // 16

    def _per_tile(v):
        v = v.reshape(NW, SPT)
        return jnp.pad(v, ((0, 0), (0, SPTB - SPT)))

    mesh = plsc.VectorSubcoreMesh(core_axis_name="c", subcore_axis_name="s")

    @functools.partial(
        pl.kernel, mesh=mesh,
        out_type=jax.ShapeDtypeStruct((B * S, nsample, D), jnp.float32),
        scratch_types=[
            pltpu.VMEM((Npad,), jnp.float32),
            pltpu.VMEM((Npad,), jnp.float32),
            pltpu.VMEM((Npad,), jnp.float32),
            pltpu.VMEM((Npad,), jnp.float32),
            pltpu.VMEM((SPTB,), jnp.float32),
            pltpu.VMEM((SPTB,), jnp.float32),
            pltpu.VMEM((SPTB,), jnp.float32),
            pltpu.VMEM((SPTB,), jnp.float32),
            pltpu.VMEM((nsample + 16,), jnp.int32),
            pltpu.VMEM((2, nsample), jnp.int32),
            pltpu.VMEM((2, nsample, D), jnp.float32),
            pltpu.SemaphoreType.DMA,
            pltpu.SemaphoreType.DMA,
        ],
        compiler_params=pltpu.CompilerParams(needs_layout_passes=False,
                                             use_tc_tiling_on_sc=False),
    )
    def bq(nx2_h, ny2_h, nz2_h, sb_h, ax_h, ay_h, az_h, sa_h, table_h, out_h,
           nx2_v, ny2_v, nz2_v, sb_v, ax_v, ay_v, az_v, sa_v,
           idx_v, gidx_v, rows_v, gsem0, osem0):
        l16 = lax.broadcasted_iota(jnp.int32, (16,), 0)
        wid = lax.axis_index("s") * 2 + lax.axis_index("c")
        b = wid // TPB
        pltpu.sync_copy(nx2_h.at[b], nx2_v)
        pltpu.sync_copy(ny2_h.at[b], ny2_v)
        pltpu.sync_copy(nz2_h.at[b], nz2_v)
        pltpu.sync_copy(sb_h.at[b], sb_v)
        pltpu.sync_copy(ax_h.at[wid], ax_v)
        pltpu.sync_copy(ay_h.at[wid], ay_v)
        pltpu.sync_copy(az_h.at[wid], az_v)
        pltpu.sync_copy(sa_h.at[wid], sa_v)

        def scan_center(c):
            cl = lax.rem(c, 16)
            gbase = c - cl
            lsel = l16 == cl
            ax_c = jnp.sum(jnp.where(lsel, ax_v[pl.ds(gbase, 16)], 0.0), axis=0)
            ay_c = jnp.sum(jnp.where(lsel, ay_v[pl.ds(gbase, 16)], 0.0), axis=0)
            az_c = jnp.sum(jnp.where(lsel, az_v[pl.ds(gbase, 16)], 0.0), axis=0)
            sa_c = jnp.sum(jnp.where(lsel, sa_v[pl.ds(gbase, 16)], 0.0), axis=0)

            def scan_body(t, cnt):
                off = t * 16
                dot2 = (ax_c * nx2_v[pl.ds(off, 16)]
                        + ay_c * ny2_v[pl.ds(off, 16)])
                dot2 = dot2 + az_c * nz2_v[pl.ds(off, 16)]
                d = dot2 + sa_c
                d = d + sb_v[pl.ds(off, 16)]
                inr = jnp.logical_not(d > r2)

                @pl.when(cnt < nsample)
                def _():
                    plsc.store_compressed(idx_v.at[pl.ds(cnt, 16)],
                                          off + l16, mask=inr)

                pc = plsc.all_reduce_population_count(inr)
                return cnt + jnp.max(pc, axis=0)

            cnt = lax.fori_loop(0, NT, scan_body, jnp.int32(0), unroll=False)
            t_eff = jnp.minimum(cnt, nsample)
            first = jnp.sum(jnp.where(l16 == 0, idx_v[pl.ds(0, 16)], 0), axis=0)
            for g in range(NS16):
                slot = g * 16 + l16
                vg = idx_v[pl.ds(g * 16, 16)]
                gidx_v[0, pl.ds(g * 16, 16)] = (
                    jnp.where(slot < t_eff, vg, first) + b * Npad)

        def center_body(c, carry):
            scan_center(c)
            pltpu.async_copy(table_h.at[gidx_v.at[0]], rows_v.at[0],
                             gsem0).wait()
            pltpu.sync_copy(rows_v.at[0], out_h.at[wid * SPT + c])
            return carry

        lax.fori_loop(0, SPT, center_body, jnp.int32(0), unroll=False)

    return bq(neg2[0], neg2[1], neg2[2], sb,
              _per_tile(cxyz[0]), _per_tile(cxyz[1]), _per_tile(cxyz[2]),
              _per_tile(sa), table)


def _gather(points, idx):
    B = idx.shape[0]
    C = points.shape[-1]
    flat = jnp.take_along_axis(points, idx.reshape(B, -1, 1), axis=1)
    return flat.reshape(idx.shape + (C,))

def _cbr(x, layer):
    W, b, gm, bt = layer
    y = jnp.einsum('...i,io->...o', x, W) + b
    axes = tuple(range(y.ndim - 1))
    mean = jnp.mean(y, axis=axes)
    var = jnp.var(y, axis=axes)
    y = (y - mean) / jnp.sqrt(var + 1e-5) * gm + bt
    return jax.nn.relu(y)

def _sa(xyz, features, npoint, radius, nsample, layers):
    B, N, _ = xyz.shape
    inds, new_xyz, neg2, sb, sa = _fps(xyz, npoint)
    Npad = sb.shape[1]
    C = 3 + (features.shape[-1] if features is not None else 0)
    D = -(-C // 16) * 16
    table = xyz if features is None else jnp.concatenate([xyz, features], -1)
    table = jnp.pad(table, ((0, 0), (0, Npad - N), (0, D - C)))
    table = table.reshape(B * Npad, D)
    cxyz = jnp.moveaxis(new_xyz, -1, 0)
    rows = _ball_group_sc(neg2, sb, cxyz, sa, table, radius,
                          npoint, nsample, D)
    rows = rows.reshape(B, npoint, nsample, D)
    gx = (rows[..., :3] - new_xyz[:, :, None, :]) / radius
    if features is not None:
        g = jnp.concatenate([gx, rows[..., 3:C]], axis=-1)
    else:
        g = gx
    for layer in layers:
        g = _cbr(g, layer)
    return new_xyz, jnp.max(g, axis=2), inds

def _fp(xyz1, xyz2, feat1, feat2, layers):
    d = _sqdist(xyz1, xyz2)
    negd, idx = lax.top_k(-d, 3)
    dist_recip = 1.0 / (-negd + 1e-8)
    weight = dist_recip / jnp.sum(dist_recip, axis=-1, keepdims=True)
    interp = jnp.sum(_gather(feat2, idx) * weight[..., None], axis=2)
    g = jnp.concatenate([interp, feat1], axis=-1)
    for layer in layers:
        g = _cbr(g, layer)
    return g


def kernel(pointcloud, params):
    xyz = pointcloud[..., 0:3]
    features = pointcloud[..., 3:]
    sa1_xyz, sa1_f, sa1_inds = _sa(xyz, features, 512, _R, 64, params['sa1'])
    sa2_xyz, sa2_f, sa2_inds = _sa(sa1_xyz, sa1_f, 256, 2 * _R, 32, params['sa2'])
    sa3_xyz, sa3_f, sa3_inds = _sa(sa2_xyz, sa2_f, 64, 4 * _R, 16, params['sa3'])
    sa4_xyz, sa4_f, sa4_inds = _sa(sa3_xyz, sa3_f, 16, 8 * _R, 16, params['sa4'])
    fp1_f = _fp(sa3_xyz, sa4_xyz, sa3_f, sa4_f, params['fp1'])
    fp2_f = _fp(sa2_xyz, sa3_xyz, sa2_f, fp1_f, params['fp2'])
    fp2_inds = sa1_inds[:, :fp2_f.shape[1]]
    return fp2_f, sa2_xyz, fp2_inds
